# initial kernel scaffold (unmeasured)
import jax
import jax.numpy as jnp
from jax import lax
from jax.experimental import pallas as pl
from jax.experimental.pallas import tpu as pltpu

N_DEV = 32
BLK = 128
K = 4096
N_OUT = 2048


def _dot_i8(a, b):
    return lax.dot_general(
        a, b,
        dimension_numbers=(((1,), (0,)), ((), ())),
        preferred_element_type=jnp.int32,
    )


def kernel(x, w_mat, scale_x, scale_w):
    m_loc, n_out = BLK, N_OUT

    def body(x_ref, w_ref, sx_ref, sw_ref, out_ref,
             comm_ref, send_sems, recv_sems):
        me = lax.axis_index("i")

        sends = []
        for s in range(1, N_DEV):
            dst = lax.rem(me + s, N_DEV)
            rdma = pltpu.make_async_remote_copy(
                src_ref=x_ref.at[pl.ds(dst * BLK, BLK), :],
                dst_ref=comm_ref.at[s],
                send_sem=send_sems.at[s],
                recv_sem=recv_sems.at[s],
                device_id=(dst,),
                device_id_type=pl.DeviceIdType.MESH,
            )
            rdma.start()
            sends.append(rdma)

        own = x_ref[pl.ds(me * BLK, BLK), :]
        w_own = w_ref[pl.ds(me * BLK, BLK), :]
        acc = _dot_i8(own, w_own)

        for s in range(1, N_DEV):
            recv = pltpu.make_async_remote_copy(
                src_ref=x_ref.at[pl.ds(0, BLK), :],
                dst_ref=comm_ref.at[s],
                send_sem=send_sems.at[s],
                recv_sem=recv_sems.at[s],
                device_id=(me,),
                device_id_type=pl.DeviceIdType.MESH,
            )
            recv.wait_recv()
            src = lax.rem(me + (N_DEV - s), N_DEV)
            w_blk = w_ref[pl.ds(src * BLK, BLK), :]
            acc = acc + _dot_i8(comm_ref[s], w_blk)

        for rdma in sends:
            rdma.wait_send()

        scale = sx_ref[0] * sw_ref[0]
        out_ref[:, :] = jnp.maximum(acc.astype(jnp.float32) * scale, 0.0)

    return pl.pallas_call(
        body,
        out_shape=jax.ShapeDtypeStruct((m_loc, n_out), jnp.float32),
        in_specs=[
            pl.BlockSpec(memory_space=pltpu.VMEM),
            pl.BlockSpec(memory_space=pltpu.VMEM),
            pl.BlockSpec(memory_space=pltpu.SMEM),
            pl.BlockSpec(memory_space=pltpu.SMEM),
        ],
        out_specs=pl.BlockSpec(memory_space=pltpu.VMEM),
        scratch_shapes=[
            pltpu.VMEM((N_DEV, BLK, BLK), jnp.int8),
            pltpu.SemaphoreType.DMA((N_DEV,)),
            pltpu.SemaphoreType.DMA((N_DEV,)),
        ],
        compiler_params=pltpu.CompilerParams(collective_id=0),
    )(x, w_mat, scale_x, scale_w)


# baseline (device time: 34855 ns/iter reference)
import jax
import jax.numpy as jnp
from jax import lax
from jax.experimental import pallas as pl
from jax.experimental.pallas import tpu as pltpu

N_DEV = 32
BLK = 128
K = 4096
N_OUT = 2048


def _dot_i8(a, b):
    return lax.dot_general(
        a, b,
        dimension_numbers=(((1,), (0,)), ((), ())),
        preferred_element_type=jnp.int32,
    )


def kernel(x, w_mat, scale_x, scale_w):
    m_loc, n_out = BLK, N_OUT

    def body(x_ref, w_ref, sx_ref, sw_ref, out_ref,
             comm_ref, send_sems, recv_sems):
        me = lax.axis_index("i")

        sends = []
        for s in range(1, N_DEV):
            dst = lax.rem(me + s, N_DEV)
            rdma = pltpu.make_async_remote_copy(
                src_ref=x_ref.at[pl.ds(dst * BLK, BLK), :],
                dst_ref=comm_ref.at[s],
                send_sem=send_sems.at[s],
                recv_sem=recv_sems.at[s],
                device_id=(dst,),
                device_id_type=pl.DeviceIdType.MESH,
            )
            rdma.start()
            sends.append(rdma)

        own = x_ref[pl.ds(me * BLK, BLK), :]
        w_own = w_ref[pl.ds(me * BLK, BLK), :]
        acc = _dot_i8(own, w_own)

        for s in range(1, N_DEV):
            recv = pltpu.make_async_remote_copy(
                src_ref=x_ref.at[pl.ds(0, BLK), :],
                dst_ref=comm_ref.at[s],
                send_sem=send_sems.at[s],
                recv_sem=recv_sems.at[s],
                device_id=(me,),
                device_id_type=pl.DeviceIdType.MESH,
            )
            recv.wait_recv()
            src = lax.rem(me + (N_DEV - s), N_DEV)
            w_blk = w_ref[pl.ds(src * BLK, BLK), :]
            acc = acc + _dot_i8(comm_ref[s], w_blk)

        for rdma in sends:
            rdma.wait_send()

        scale = sx_ref[0] * sw_ref[0]
        out_ref[:, :] = jnp.maximum(acc.astype(jnp.float32) * scale, 0.0)

    return pl.pallas_call(
        body,
        out_shape=jax.ShapeDtypeStruct((m_loc, n_out), jnp.float32),
        in_specs=[
            pl.BlockSpec(memory_space=pltpu.VMEM),
            pl.BlockSpec(memory_space=pltpu.VMEM),
            pl.BlockSpec(memory_space=pltpu.SMEM),
            pl.BlockSpec(memory_space=pltpu.SMEM),
        ],
        out_specs=pl.BlockSpec(memory_space=pltpu.VMEM),
        scratch_shapes=[
            pltpu.VMEM((N_DEV, BLK, BLK), jnp.int8),
            pltpu.SemaphoreType.DMA((N_DEV,)),
            pltpu.SemaphoreType.DMA((N_DEV,)),
        ],
    )(x, w_mat, scale_x, scale_w)


# device time: 27907 ns/iter; 1.2490x vs baseline; 1.2490x over previous
import jax
import jax.numpy as jnp
from jax import lax
from jax.experimental import pallas as pl
from jax.experimental.pallas import tpu as pltpu

N_DEV = 32
BLK = 128
K = 4096
N_OUT = 2048


def kernel(x, w_mat, scale_x, scale_w):
    def body(x_ref, w_ref, sx_ref, sw_ref, out_ref,
             xfull_ref, send_sems, recv_sems):
        me = lax.axis_index("i")

        sends = []
        for s in range(1, N_DEV):
            dst = lax.rem(me + s, N_DEV)
            rdma = pltpu.make_async_remote_copy(
                src_ref=x_ref.at[pl.ds(dst * BLK, BLK), :],
                dst_ref=xfull_ref.at[:, pl.ds(me * BLK, BLK)],
                send_sem=send_sems.at[s],
                recv_sem=recv_sems.at[s],
                device_id=(dst,),
                device_id_type=pl.DeviceIdType.MESH,
            )
            rdma.start()
            sends.append(rdma)

        xfull_ref[:, pl.ds(me * BLK, BLK)] = x_ref[pl.ds(me * BLK, BLK), :]

        for s in range(1, N_DEV):
            recv = pltpu.make_async_remote_copy(
                src_ref=x_ref.at[pl.ds(0, BLK), :],
                dst_ref=xfull_ref.at[:, pl.ds(0, BLK)],
                send_sem=send_sems.at[s],
                recv_sem=recv_sems.at[s],
                device_id=(me,),
                device_id_type=pl.DeviceIdType.MESH,
            )
            recv.wait_recv()

        acc = lax.dot_general(
            xfull_ref[:, :], w_ref[:, :],
            dimension_numbers=(((1,), (0,)), ((), ())),
            preferred_element_type=jnp.int32,
        )

        for rdma in sends:
            rdma.wait_send()

        scale = sx_ref[0] * sw_ref[0]
        out_ref[:, :] = jnp.maximum(acc.astype(jnp.float32) * scale, 0.0)

    return pl.pallas_call(
        body,
        out_shape=jax.ShapeDtypeStruct((BLK, N_OUT), jnp.float32),
        in_specs=[
            pl.BlockSpec(memory_space=pltpu.VMEM),
            pl.BlockSpec(memory_space=pltpu.VMEM),
            pl.BlockSpec(memory_space=pltpu.SMEM),
            pl.BlockSpec(memory_space=pltpu.SMEM),
        ],
        out_specs=pl.BlockSpec(memory_space=pltpu.VMEM),
        scratch_shapes=[
            pltpu.VMEM((BLK, K), jnp.int8),
            pltpu.SemaphoreType.DMA((N_DEV,)),
            pltpu.SemaphoreType.DMA((N_DEV,)),
        ],
    )(x, w_mat, scale_x, scale_w)


# device time: 24027 ns/iter; 1.4507x vs baseline; 1.1615x over previous
import jax
import jax.numpy as jnp
from jax import lax
from jax.experimental import pallas as pl
from jax.experimental.pallas import tpu as pltpu

N_DEV = 32
BLK = 128
K = 4096
N_OUT = 2048
HALF = N_DEV // 2


def kernel(x, w_mat, scale_x, scale_w):
    def body(x_ref, w_ref, sx_ref, sw_ref, out_ref,
             xfull_ref, send_sems, recv_sems, credit_sems):
        me = lax.axis_index("i")

        barrier_sem = pltpu.get_barrier_semaphore()
        pl.semaphore_signal(barrier_sem, inc=1)
        pl.semaphore_wait(barrier_sem, 1)

        for s in range(1, N_DEV):
            src = lax.rem(me + (N_DEV - s), N_DEV)
            pl.semaphore_signal(
                credit_sems.at[s], inc=1,
                device_id=(src,), device_id_type=pl.DeviceIdType.MESH,
            )

        xfull_ref[:, pl.ds(me * BLK, BLK)] = x_ref[pl.ds(me * BLK, BLK), :]

        sends = []
        for s in range(1, N_DEV):
            dst = lax.rem(me + s, N_DEV)
            pl.semaphore_wait(credit_sems.at[s], 1)
            rdma = pltpu.make_async_remote_copy(
                src_ref=x_ref.at[pl.ds(dst * BLK, BLK), :],
                dst_ref=xfull_ref.at[:, pl.ds(me * BLK, BLK)],
                send_sem=send_sems.at[s],
                recv_sem=recv_sems.at[me],
                device_id=(dst,),
                device_id_type=pl.DeviceIdType.MESH,
            )
            rdma.start()
            sends.append(rdma)

        def wait_src(j):
            @pl.when(j != me)
            def _():
                recv = pltpu.make_async_remote_copy(
                    src_ref=x_ref.at[pl.ds(0, BLK), :],
                    dst_ref=xfull_ref.at[:, pl.ds(0, BLK)],
                    send_sem=send_sems.at[0],
                    recv_sem=recv_sems.at[j],
                    device_id=(me,),
                    device_id_type=pl.DeviceIdType.MESH,
                )
                recv.wait_recv()

        for j in range(HALF):
            wait_src(j)
        acc = lax.dot_general(
            xfull_ref[:, 0:HALF * BLK], w_ref[0:HALF * BLK, :],
            dimension_numbers=(((1,), (0,)), ((), ())),
            preferred_element_type=jnp.int32,
        )
        for j in range(HALF, N_DEV):
            wait_src(j)
        acc = acc + lax.dot_general(
            xfull_ref[:, HALF * BLK:K], w_ref[HALF * BLK:K, :],
            dimension_numbers=(((1,), (0,)), ((), ())),
            preferred_element_type=jnp.int32,
        )

        for rdma in sends:
            rdma.wait_send()

        scale = sx_ref[0] * sw_ref[0]
        out_ref[:, :] = jnp.maximum(acc.astype(jnp.float32) * scale, 0.0)

    return pl.pallas_call(
        body,
        out_shape=jax.ShapeDtypeStruct((BLK, N_OUT), jnp.float32),
        in_specs=[
            pl.BlockSpec(memory_space=pltpu.VMEM),
            pl.BlockSpec(memory_space=pltpu.VMEM),
            pl.BlockSpec(memory_space=pltpu.SMEM),
            pl.BlockSpec(memory_space=pltpu.SMEM),
        ],
        out_specs=pl.BlockSpec(memory_space=pltpu.VMEM),
        scratch_shapes=[
            pltpu.VMEM((BLK, K), jnp.int8),
            pltpu.SemaphoreType.DMA((N_DEV,)),
            pltpu.SemaphoreType.DMA((N_DEV,)),
            pltpu.SemaphoreType.REGULAR((N_DEV,)),
        ],
        compiler_params=pltpu.CompilerParams(collective_id=0),
    )(x, w_mat, scale_x, scale_w)


# device time: 23155 ns/iter; 1.5053x vs baseline; 1.0377x over previous
import jax
import jax.numpy as jnp
from jax import lax
from jax.experimental import pallas as pl
from jax.experimental.pallas import tpu as pltpu

N_DEV = 32
BLK = 128
K = 4096
N_OUT = 2048
Q = N_DEV // 4

SEND_ORDER = []
for d in range(1, N_DEV // 2 + 1):
    SEND_ORDER.append(d)
    if d != N_DEV - d:
        SEND_ORDER.append(N_DEV - d)


def kernel(x, w_mat, scale_x, scale_w):
    def body(x_ref, w_ref, sx_ref, sw_ref, out_ref,
             xfull_ref, send_sems, recv_sems, credit_sems):
        me = lax.axis_index("i")

        barrier_sem = pltpu.get_barrier_semaphore()
        pl.semaphore_signal(barrier_sem, inc=1)
        pl.semaphore_wait(barrier_sem, 1)

        for s in SEND_ORDER:
            src = lax.rem(me + (N_DEV - s), N_DEV)
            pl.semaphore_signal(
                credit_sems.at[s], inc=1,
                device_id=(src,), device_id_type=pl.DeviceIdType.MESH,
            )

        xfull_ref[:, pl.ds(me * BLK, BLK)] = x_ref[pl.ds(me * BLK, BLK), :]

        sends = []
        for s in SEND_ORDER:
            dst = lax.rem(me + s, N_DEV)
            pl.semaphore_wait(credit_sems.at[s], 1)
            rdma = pltpu.make_async_remote_copy(
                src_ref=x_ref.at[pl.ds(dst * BLK, BLK), :],
                dst_ref=xfull_ref.at[:, pl.ds(me * BLK, BLK)],
                send_sem=send_sems.at[s],
                recv_sem=recv_sems.at[me],
                device_id=(dst,),
                device_id_type=pl.DeviceIdType.MESH,
            )
            rdma.start()
            sends.append(rdma)

        def wait_src(j):
            @pl.when(j != me)
            def _():
                recv = pltpu.make_async_remote_copy(
                    src_ref=x_ref.at[pl.ds(0, BLK), :],
                    dst_ref=xfull_ref.at[:, pl.ds(0, BLK)],
                    send_sem=send_sems.at[0],
                    recv_sem=recv_sems.at[j],
                    device_id=(me,),
                    device_id_type=pl.DeviceIdType.MESH,
                )
                recv.wait_recv()

        acc = None
        for p in range(4):
            for j in range(p * Q, (p + 1) * Q):
                wait_src(j)
            part = lax.dot_general(
                xfull_ref[:, p * Q * BLK:(p + 1) * Q * BLK],
                w_ref[p * Q * BLK:(p + 1) * Q * BLK, :],
                dimension_numbers=(((1,), (0,)), ((), ())),
                preferred_element_type=jnp.int32,
            )
            acc = part if acc is None else acc + part

        for rdma in sends:
            rdma.wait_send()

        scale = sx_ref[0] * sw_ref[0]
        out_ref[:, :] = jnp.maximum(acc.astype(jnp.float32) * scale, 0.0)

    return pl.pallas_call(
        body,
        out_shape=jax.ShapeDtypeStruct((BLK, N_OUT), jnp.float32),
        in_specs=[
            pl.BlockSpec(memory_space=pltpu.VMEM),
            pl.BlockSpec(memory_space=pltpu.VMEM),
            pl.BlockSpec(memory_space=pltpu.SMEM),
            pl.BlockSpec(memory_space=pltpu.SMEM),
        ],
        out_specs=pl.BlockSpec(memory_space=pltpu.VMEM),
        scratch_shapes=[
            pltpu.VMEM((BLK, K), jnp.int8),
            pltpu.SemaphoreType.DMA((N_DEV,)),
            pltpu.SemaphoreType.DMA((N_DEV,)),
            pltpu.SemaphoreType.REGULAR((N_DEV,)),
        ],
        compiler_params=pltpu.CompilerParams(collective_id=0),
    )(x, w_mat, scale_x, scale_w)
